# baseline (speedup 1.0000x reference)
import functools

import jax
import jax.numpy as jnp
from jax import lax
from jax.experimental import pallas as pl
from jax.experimental.pallas import tpu as pltpu
from jax.experimental.pallas import tpu_sc as plsc

BATCH = 16384
D = 16
VOC = 1_000_000

_info = plsc.get_sparse_core_info()
NC = _info.num_cores
NS = _info.num_subcores
L = _info.num_lanes
NW = NC * NS
BPW = BATCH // NW
CHUNK = 64
NCHUNK = BPW // CHUNK
VECS = CHUNK // L

_mesh = plsc.VectorSubcoreMesh(core_axis_name="c", subcore_axis_name="s")


@functools.partial(
    pl.kernel,
    mesh=_mesh,
    out_type=jax.ShapeDtypeStruct((BATCH,), jnp.float32),
    scratch_types=[
        pltpu.VMEM((BPW,), jnp.int32),
        pltpu.VMEM((BPW,), jnp.int32),
        pltpu.VMEM((D, CHUNK), jnp.int32),
        pltpu.VMEM((D, CHUNK), jnp.int32),
        pltpu.VMEM((2, D, CHUNK, L), jnp.float32),
        pltpu.VMEM((2, D, CHUNK, L), jnp.float32),
        pltpu.VMEM((BPW,), jnp.float32),
        pltpu.SemaphoreType.DMA,
        pltpu.SemaphoreType.DMA,
    ],
    compiler_params=pltpu.CompilerParams(
        needs_layout_passes=False, use_tc_tiling_on_sc=False
    ),
)
def _pmf_sc(uidx_hbm, iidx_hbm, wut_hbm, wit_hbm, out_hbm,
            uidx_v, iidx_v, ugr_v, igr_v, ubuf_v, ibuf_v, out_v, usem, isem):
    wid = lax.axis_index("s") * NC + lax.axis_index("c")
    base = wid * BPW

    pltpu.sync_copy(uidx_hbm.at[pl.ds(base, BPW)], uidx_v)
    pltpu.sync_copy(iidx_hbm.at[pl.ds(base, BPW)], iidx_v)

    def prep_chunk(c):
        for v in range(VECS):
            s = pl.ds(c * CHUNK + v * L, L)
            t = pl.ds(v * L, L)
            ug = lax.shift_right_logical(uidx_v[s], 4)
            ig = lax.shift_right_logical(iidx_v[s], 4)
            for d in range(D):
                ugr_v[d, t] = ug
                igr_v[d, t] = ig

    def start_chunk(c):
        b = c % 2
        ucopies = []
        icopies = []
        for d in range(D):
            ucopies.append(pltpu.async_copy(
                wut_hbm.at[d].at[ugr_v.at[d]], ubuf_v.at[b, d], usem))
            icopies.append(pltpu.async_copy(
                wit_hbm.at[d].at[igr_v.at[d]], ibuf_v.at[b, d], isem))
        return ucopies, icopies

    lane = lax.iota(jnp.int32, L)
    prep_chunk(0)
    inflight = start_chunk(0)

    for c in range(NCHUNK):
        ucopies, icopies = inflight
        for cp in ucopies:
            cp.wait()
        for cp in icopies:
            cp.wait()
        if c + 1 < NCHUNK:
            prep_chunk(c + 1)
            inflight = start_chunk(c + 1)
        b = c % 2

        def group_body(g, _, c=c, b=b):
            s = pl.ds(c * CHUNK + g * L, L)
            rows = g * L + lane
            uoff = uidx_v[s] & (L - 1)
            ioff = iidx_v[s] & (L - 1)
            acc = jnp.zeros((L,), jnp.float32)
            for d in range(D):
                dv = jnp.full((L,), d, jnp.int32)
                bv = jnp.full((L,), b, jnp.int32)
                uc = plsc.load_gather(ubuf_v, [bv, dv, rows, uoff])
                ic = plsc.load_gather(ibuf_v, [bv, dv, rows, ioff])
                acc = acc + uc * ic
            out_v[pl.ds(c * CHUNK + g * L, L)] = acc
            return 0

        lax.fori_loop(0, VECS, group_body, 0)

    pltpu.sync_copy(out_v, out_hbm.at[pl.ds(base, BPW)])


def kernel(uesr_indices, item_indices, w_user, w_item):
    uidx = uesr_indices.astype(jnp.int32)
    iidx = item_indices.astype(jnp.int32)
    wut = w_user.T.reshape(D, VOC // L, L)
    wit = w_item.T.reshape(D, VOC // L, L)
    return _pmf_sc(uidx, iidx, wut, wit)


# speedup vs baseline: 3.1875x; 3.1875x over previous
import functools

import jax
import jax.numpy as jnp
from jax import lax
from jax.experimental import pallas as pl
from jax.experimental.pallas import tpu as pltpu
from jax.experimental.pallas import tpu_sc as plsc

BATCH = 16384
D = 16

_info = plsc.get_sparse_core_info()
NC = _info.num_cores
NS = _info.num_subcores
L = _info.num_lanes
NW = NC * NS
BPW = BATCH // NW
CHUNK = 128
NCHUNK = BPW // CHUNK
GROUPS = BPW // L

_mesh = plsc.VectorSubcoreMesh(core_axis_name="c", subcore_axis_name="s")


@functools.partial(
    pl.kernel,
    mesh=_mesh,
    out_type=jax.ShapeDtypeStruct((BATCH,), jnp.float32),
    scratch_types=[
        pltpu.VMEM((NCHUNK, CHUNK), jnp.int32),
        pltpu.VMEM((NCHUNK, CHUNK), jnp.int32),
        pltpu.VMEM((BPW, D), jnp.float32),
        pltpu.VMEM((BPW, D), jnp.float32),
        pltpu.VMEM((BPW,), jnp.float32),
        pltpu.SemaphoreType.DMA,
        pltpu.SemaphoreType.DMA,
    ],
    compiler_params=pltpu.CompilerParams(
        needs_layout_passes=False, use_tc_tiling_on_sc=False
    ),
)
def _pmf_sc(uidx_hbm, iidx_hbm, wu_hbm, wi_hbm, out_hbm,
            uidx_v, iidx_v, urows_v, irows_v, out_v, usem, isem):
    wid = lax.axis_index("s") * NC + lax.axis_index("c")
    base_row = wid * NCHUNK

    pltpu.sync_copy(uidx_hbm.at[pl.ds(base_row, NCHUNK)], uidx_v)
    pltpu.sync_copy(iidx_hbm.at[pl.ds(base_row, NCHUNK)], iidx_v)

    ucopies = []
    icopies = []
    for c in range(NCHUNK):
        dst_u = urows_v.at[pl.ds(c * CHUNK, CHUNK), :]
        dst_i = irows_v.at[pl.ds(c * CHUNK, CHUNK), :]
        ucopies.append(pltpu.async_copy(wu_hbm.at[uidx_v.at[c]], dst_u, usem))
        icopies.append(pltpu.async_copy(wi_hbm.at[iidx_v.at[c]], dst_i, isem))
    for cp in ucopies:
        cp.wait()
    for cp in icopies:
        cp.wait()

    lane = lax.iota(jnp.int32, L)

    def group_body(g, _):
        row0 = g * L
        row_idx = row0 + lane
        acc = jnp.zeros((L,), jnp.float32)
        for d in range(D):
            col_idx = jnp.full((L,), d, jnp.int32)
            uc = plsc.load_gather(urows_v, [row_idx, col_idx])
            ic = plsc.load_gather(irows_v, [row_idx, col_idx])
            acc = acc + uc * ic
        out_v[pl.ds(row0, L)] = acc
        return 0

    lax.fori_loop(0, GROUPS, group_body, 0)

    pltpu.sync_copy(out_v, out_hbm.at[pl.ds(wid * BPW, BPW)])


def kernel(uesr_indices, item_indices, w_user, w_item):
    uidx = uesr_indices.astype(jnp.int32).reshape(NW * NCHUNK, CHUNK)
    iidx = item_indices.astype(jnp.int32).reshape(NW * NCHUNK, CHUNK)
    return _pmf_sc(uidx, iidx, w_user, w_item)
